# baseline (speedup 1.0000x reference)
import functools

import jax
import jax.numpy as jnp
from jax import lax
from jax.experimental import pallas as pl
from jax.experimental.pallas import tpu as pltpu
from jax.experimental.pallas import tpu_sc as plsc


def _make_gather(N, C, EOFF, EH):
    NW = 32
    EW = EH // NW
    BE = 40
    NB = EW // BE
    mesh = plsc.VectorSubcoreMesh(core_axis_name="c", subcore_axis_name="s")

    @functools.partial(
        pl.kernel,
        out_type=jax.ShapeDtypeStruct((EH, C), jnp.float32),
        mesh=mesh,
        scratch_types=[
            pltpu.VMEM((EW,), jnp.int32),
            pltpu.VMEM((EW,), jnp.int32),
            pltpu.VMEM((BE, C), jnp.float32),
            pltpu.VMEM((BE, C), jnp.float32),
            pltpu.VMEM((BE, C), jnp.float32),
            pltpu.VMEM((BE, C), jnp.float32),
            pltpu.SemaphoreType.DMA,
            pltpu.SemaphoreType.DMA,
            pltpu.SemaphoreType.DMA,
            pltpu.SemaphoreType.DMA,
        ],
    )
    def gather_k(cat_hbm, pi_hbm, pj_hbm, s_hbm, idx_ia, idx_ja,
                 ri0, rj0, ri1, rj1, si0, sj0, si1, sj1):
        cid = lax.axis_index("c")
        sid = lax.axis_index("s")
        wid = sid * 2 + cid
        wbase = wid * EW
        pltpu.sync_copy(pi_hbm.at[pl.ds(EOFF + wbase, EW)], idx_ia)
        pltpu.sync_copy(pj_hbm.at[pl.ds(EOFF + wbase, EW)], idx_ja)

        def fire(b, ri, rj, si, sj):
            pltpu.async_copy(cat_hbm.at[idx_ia.at[pl.ds(b * BE, BE)]], ri, si)
            pltpu.async_copy(cat_hbm.at[idx_ja.at[pl.ds(b * BE, BE)]], rj, sj)

        def finish(b, ri, rj, si, sj):
            pltpu.make_async_copy(
                cat_hbm.at[idx_ia.at[pl.ds(b * BE, BE)]], ri, si).wait()
            pltpu.make_async_copy(
                cat_hbm.at[idx_ja.at[pl.ds(b * BE, BE)]], rj, sj).wait()

            def add_row(e, c2):
                for g in range(C // 16):
                    sl = pl.ds(g * 16, 16)
                    ri[e, sl] = ri[e, sl] + rj[e, sl]
                return c2

            lax.fori_loop(0, BE, add_row, 0)
            pltpu.sync_copy(ri, s_hbm.at[pl.ds(wbase + b * BE, BE)])

        fire(0, ri0, rj0, si0, sj0)
        L = (NB - 1) // 2

        def body(b2, carry):
            b0 = 2 * b2
            fire(b0 + 1, ri1, rj1, si1, sj1)
            finish(b0, ri0, rj0, si0, sj0)
            fire(b0 + 2, ri0, rj0, si0, sj0)
            finish(b0 + 1, ri1, rj1, si1, sj1)
            return carry

        lax.fori_loop(0, L, body, 0)
        if NB % 2 == 1:
            finish(2 * L, ri0, rj0, si0, sj0)
        else:
            fire(2 * L + 1, ri1, rj1, si1, sj1)
            finish(2 * L, ri0, rj0, si0, sj0)
            finish(2 * L + 1, ri1, rj1, si1, sj1)

    return gather_k


def _make_edge(EH, F, B, OFFB):
    Eb = 640
    grid = EH // Eb
    C = 4 * F

    def body(s_ref, basis_ref, diff_ref, wpi_ref, wii_ref, wpix_ref,
             i1_ref, ix3_ref):
        s1 = s_ref[:, :F]
        inter = jnp.tanh(
            jnp.dot(s1, wpi_ref[...], preferred_element_type=jnp.float32))
        u = inter[:, 0:F] * basis_ref[:, 0:1]
        for b in range(1, B):
            u = u + inter[:, b * F:(b + 1) * F] * basis_ref[:, b:b + 1]
        i1 = jnp.tanh(
            jnp.dot(u, wii_ref[...], preferred_element_type=jnp.float32))
        i1_ref[...] = i1
        for x in range(3):
            sx = s_ref[:, (x + 1) * F:(x + 2) * F]
            t = jnp.dot(sx, wpix_ref[...], preferred_element_type=jnp.float32)
            ix3_ref[x, :, :] = (t + diff_ref[:, x:x + 1]) * i1

    return pl.pallas_call(
        body,
        grid=(grid,),
        in_specs=[
            pl.BlockSpec((Eb, C), lambda i: (i, 0)),
            pl.BlockSpec((Eb, B), lambda i: (i + OFFB, 0)),
            pl.BlockSpec((Eb, 3), lambda i: (i + OFFB, 0)),
            pl.BlockSpec((F, F * B), lambda i: (0, 0)),
            pl.BlockSpec((F, F), lambda i: (0, 0)),
            pl.BlockSpec((F, F), lambda i: (0, 0)),
        ],
        out_specs=[
            pl.BlockSpec((Eb, F), lambda i: (i, 0)),
            pl.BlockSpec((3, Eb, F), lambda i: (0, i, 0)),
        ],
        out_shape=[
            jax.ShapeDtypeStruct((EH, F), jnp.float32),
            jax.ShapeDtypeStruct((3, EH, F), jnp.float32),
        ],
    )


def _make_scatter(N, F, EOFF, EH):
    ET = EH // 16
    BE = 80
    NB = ET // BE
    NP = 80
    NPc = N // NP
    mesh = plsc.VectorSubcoreMesh(core_axis_name="c", subcore_axis_name="s")

    @functools.partial(
        pl.kernel,
        out_type=[
            jax.ShapeDtypeStruct((N, F), jnp.float32),
            jax.ShapeDtypeStruct((N, 3 * F), jnp.float32),
        ],
        mesh=mesh,
        scratch_types=[
            pltpu.VMEM((BE,), jnp.int32),
            pltpu.VMEM((BE,), jnp.int32),
            pltpu.VMEM((BE, F), jnp.float32),
            pltpu.VMEM((BE, F), jnp.float32),
            pltpu.VMEM((NP, F), jnp.float32),
            pltpu.VMEM((NP, F), jnp.float32),
            pltpu.VMEM_SHARED((N, F), jnp.float32),
            pltpu.SemaphoreType.DMA,
            pltpu.SemaphoreType.DMA,
            pltpu.SemaphoreType.DMA,
            pltpu.SemaphoreType.DMA,
        ],
    )
    def scatter_k(i1_hbm, ix3_hbm, pairi_hbm, zeros_hbm, out1_hbm, out3_hbm,
                  idx0, idx1, r0b, r1b, zbuf, wbuf, acc_sh,
                  sI0, sR0, sI1, sR1):
        cid = lax.axis_index("c")
        sid = lax.axis_index("s")
        pltpu.sync_copy(zeros_hbm, zbuf)

        npieces = (NPc - sid + 15) // 16

        def run_pass(src_at, dst_at):
            def zero_piece(k, carry):
                r0 = (sid + 16 * k) * NP
                pltpu.sync_copy(zbuf, acc_sh.at[pl.ds(r0, NP)])
                return carry

            lax.fori_loop(0, npieces, zero_piece, 0)
            plsc.subcore_barrier()

            def fire(b, idx_v, rows_v, sI, sR):
                base = sid * ET + b * BE
                pltpu.async_copy(
                    pairi_hbm.at[pl.ds(EOFF + base, BE)], idx_v, sI)
                pltpu.async_copy(src_at(base), rows_v, sR)

            def finish(b, idx_v, rows_v, sI, sR):
                base = sid * ET + b * BE
                pltpu.make_async_copy(
                    pairi_hbm.at[pl.ds(EOFF + base, BE)], idx_v, sI).wait()
                pltpu.make_async_copy(src_at(base), rows_v, sR).wait()
                pltpu.sync_copy(rows_v, acc_sh.at[idx_v], add=True)

            fire(0, idx0, r0b, sI0, sR0)
            L = (NB - 1) // 2

            def blk(b2, carry):
                b0 = 2 * b2
                fire(b0 + 1, idx1, r1b, sI1, sR1)
                finish(b0, idx0, r0b, sI0, sR0)
                fire(b0 + 2, idx0, r0b, sI0, sR0)
                finish(b0 + 1, idx1, r1b, sI1, sR1)
                return carry

            lax.fori_loop(0, L, blk, 0)
            if NB % 2 == 1:
                finish(2 * L, idx0, r0b, sI0, sR0)
            else:
                fire(2 * L + 1, idx1, r1b, sI1, sR1)
                finish(2 * L, idx0, r0b, sI0, sR0)
                finish(2 * L + 1, idx1, r1b, sI1, sR1)
            plsc.subcore_barrier()

            def write_piece(k, carry):
                r0 = (sid + 16 * k) * NP
                pltpu.sync_copy(acc_sh.at[pl.ds(r0, NP)], wbuf)
                pltpu.sync_copy(wbuf, dst_at(r0))
                return carry

            lax.fori_loop(0, npieces, write_piece, 0)

        @pl.when(cid == 0)
        def _():
            run_pass(lambda b: i1_hbm.at[pl.ds(b, BE)],
                     lambda r: out1_hbm.at[pl.ds(r, NP)])
            run_pass(lambda b: ix3_hbm.at[0, pl.ds(b, BE), :],
                     lambda r: out3_hbm.at[pl.ds(r, NP), pl.ds(0, F)])

        @pl.when(cid == 1)
        def _():
            run_pass(lambda b: ix3_hbm.at[1, pl.ds(b, BE), :],
                     lambda r: out3_hbm.at[pl.ds(r, NP), pl.ds(F, F)])
            run_pass(lambda b: ix3_hbm.at[2, pl.ds(b, BE), :],
                     lambda r: out3_hbm.at[pl.ds(r, NP), pl.ds(2 * F, F)])

    return scatter_k


def _make_node(N, F):
    Nb = 2000
    grid = N // Nb

    def body(a1a_ref, a1b_ref, a3a_ref, a3b_ref, wpp_ref, bpp_ref, weq_ref,
             wout_ref, bout_ref, p1t1_ref, p3t1_ref):
        p1n = jnp.tanh(
            jnp.dot(a1a_ref[...] + a1b_ref[...], wpp_ref[...],
                    preferred_element_type=jnp.float32) + bpp_ref[...])
        p1t1_ref[:, 0, :] = jnp.dot(
            p1n, wout_ref[...], preferred_element_type=jnp.float32) + bout_ref[...]
        p3n = [
            jnp.dot(a3a_ref[:, x * F:(x + 1) * F] + a3b_ref[:, x * F:(x + 1) * F],
                    weq_ref[...], preferred_element_type=jnp.float32)
            for x in range(3)
        ]
        dot = p3n[0] * p3n[0] + p3n[1] * p3n[1] + p3n[2] * p3n[2]
        scale = jnp.dot(
            dot, wout_ref[...], preferred_element_type=jnp.float32) + bout_ref[...]
        for x in range(3):
            p3t1_ref[:, x, :] = p3n[x] * scale

    return pl.pallas_call(
        body,
        grid=(grid,),
        in_specs=[
            pl.BlockSpec((Nb, F), lambda i: (i, 0)),
            pl.BlockSpec((Nb, F), lambda i: (i, 0)),
            pl.BlockSpec((Nb, 3 * F), lambda i: (i, 0)),
            pl.BlockSpec((Nb, 3 * F), lambda i: (i, 0)),
            pl.BlockSpec((F, F), lambda i: (0, 0)),
            pl.BlockSpec((1, F), lambda i: (0, 0)),
            pl.BlockSpec((F, F), lambda i: (0, 0)),
            pl.BlockSpec((F, F), lambda i: (0, 0)),
            pl.BlockSpec((1, F), lambda i: (0, 0)),
        ],
        out_specs=[
            pl.BlockSpec((Nb, 1, F), lambda i: (i, 0, 0)),
            pl.BlockSpec((Nb, 3, F), lambda i: (i, 0, 0)),
        ],
        out_shape=[
            jax.ShapeDtypeStruct((N, 1, F), jnp.float32),
            jax.ShapeDtypeStruct((N, 3, F), jnp.float32),
        ],
    )


def kernel(p1, p3, pair_i, pair_j, basis, diff, W_pp, b_pp, W_pi, W_ii,
           W_eq_pp, W_pix, W_out, b_out):
    N, _, F = p1.shape
    E = pair_i.shape[0]
    B = basis.shape[1]

    EH0 = 81920
    EH1 = E - EH0

    cat = jnp.concatenate([p1.reshape(N, F), p3.reshape(N, 3 * F)], axis=1)
    W_pi_perm = W_pi.reshape(F, F, B).transpose(0, 2, 1).reshape(F, F * B)
    zeros = jnp.zeros((80, F), jnp.float32)

    s0 = _make_gather(N, 4 * F, 0, EH0)(cat, pair_i, pair_j)
    s1 = _make_gather(N, 4 * F, EH0, EH1)(cat, pair_i, pair_j)

    i1f_0, ix3_0 = _make_edge(EH0, F, B, 0)(
        s0, basis, diff, W_pi_perm, W_ii, W_pix)
    i1f_1, ix3_1 = _make_edge(EH1, F, B, EH0 // 640)(
        s1, basis, diff, W_pi_perm, W_ii, W_pix)

    acc1_0, acc3_0 = _make_scatter(N, F, 0, EH0)(i1f_0, ix3_0, pair_i, zeros)
    acc1_1, acc3_1 = _make_scatter(N, F, EH0, EH1)(i1f_1, ix3_1, pair_i, zeros)

    p1t1, p3t1 = _make_node(N, F)(
        acc1_0, acc1_1, acc3_0, acc3_1, W_pp, b_pp.reshape(1, F), W_eq_pp,
        W_out, b_out.reshape(1, F))

    i1 = jnp.concatenate([i1f_0, i1f_1], axis=0).reshape(E, 1, F)
    ix = jnp.concatenate([ix3_0, ix3_1], axis=1).transpose(1, 0, 2)
    return (p1t1, p3t1, i1, ix)


# speedup vs baseline: 1.0243x; 1.0243x over previous
import functools

import jax
import jax.numpy as jnp
from jax import lax
from jax.experimental import pallas as pl
from jax.experimental.pallas import tpu as pltpu
from jax.experimental.pallas import tpu_sc as plsc


def _make_gather(N, C, EOFF, EH):
    NW = 32
    EW = EH // NW
    BE = 40
    NB = EW // BE
    mesh = plsc.VectorSubcoreMesh(core_axis_name="c", subcore_axis_name="s")

    @functools.partial(
        pl.kernel,
        out_type=jax.ShapeDtypeStruct((EH, C), jnp.float32),
        mesh=mesh,
        scratch_types=[
            pltpu.VMEM((EW,), jnp.int32),
            pltpu.VMEM((EW,), jnp.int32),
            pltpu.VMEM((BE, C), jnp.float32),
            pltpu.VMEM((BE, C), jnp.float32),
            pltpu.VMEM((BE, C), jnp.float32),
            pltpu.VMEM((BE, C), jnp.float32),
            pltpu.SemaphoreType.DMA,
            pltpu.SemaphoreType.DMA,
            pltpu.SemaphoreType.DMA,
            pltpu.SemaphoreType.DMA,
        ],
    )
    def gather_k(cat_hbm, pi_hbm, pj_hbm, s_hbm, idx_ia, idx_ja,
                 ri0, rj0, ri1, rj1, si0, sj0, si1, sj1):
        cid = lax.axis_index("c")
        sid = lax.axis_index("s")
        wid = sid * 2 + cid
        wbase = wid * EW
        pltpu.sync_copy(pi_hbm.at[pl.ds(EOFF + wbase, EW)], idx_ia)
        pltpu.sync_copy(pj_hbm.at[pl.ds(EOFF + wbase, EW)], idx_ja)

        def fire(b, ri, rj, si, sj):
            pltpu.async_copy(cat_hbm.at[idx_ia.at[pl.ds(b * BE, BE)]], ri, si)
            pltpu.async_copy(cat_hbm.at[idx_ja.at[pl.ds(b * BE, BE)]], rj, sj)

        def finish(b, ri, rj, si, sj):
            pltpu.make_async_copy(
                cat_hbm.at[idx_ia.at[pl.ds(b * BE, BE)]], ri, si).wait()
            pltpu.make_async_copy(
                cat_hbm.at[idx_ja.at[pl.ds(b * BE, BE)]], rj, sj).wait()

            def add_row(e, c2):
                for g in range(C // 16):
                    sl = pl.ds(g * 16, 16)
                    ri[e, sl] = ri[e, sl] + rj[e, sl]
                return c2

            lax.fori_loop(0, BE, add_row, 0)
            pltpu.sync_copy(ri, s_hbm.at[pl.ds(wbase + b * BE, BE)])

        fire(0, ri0, rj0, si0, sj0)
        L = (NB - 1) // 2

        def body(b2, carry):
            b0 = 2 * b2
            fire(b0 + 1, ri1, rj1, si1, sj1)
            finish(b0, ri0, rj0, si0, sj0)
            fire(b0 + 2, ri0, rj0, si0, sj0)
            finish(b0 + 1, ri1, rj1, si1, sj1)
            return carry

        lax.fori_loop(0, L, body, 0)
        if NB % 2 == 1:
            finish(2 * L, ri0, rj0, si0, sj0)
        else:
            fire(2 * L + 1, ri1, rj1, si1, sj1)
            finish(2 * L, ri0, rj0, si0, sj0)
            finish(2 * L + 1, ri1, rj1, si1, sj1)

    return gather_k


def _make_edge(EH, F, B, OFFB):
    Eb = 640
    grid = EH // Eb
    C = 4 * F

    def body(s_ref, basis_ref, diff_ref, wpi_ref, wii_ref, wpix_ref,
             i1_ref, ix3_ref):
        s1 = s_ref[:, :F]
        inter = jnp.tanh(
            jnp.dot(s1, wpi_ref[...], preferred_element_type=jnp.float32))
        u = inter[:, 0:F] * basis_ref[:, 0:1]
        for b in range(1, B):
            u = u + inter[:, b * F:(b + 1) * F] * basis_ref[:, b:b + 1]
        i1 = jnp.tanh(
            jnp.dot(u, wii_ref[...], preferred_element_type=jnp.float32))
        i1_ref[...] = i1
        for x in range(3):
            sx = s_ref[:, (x + 1) * F:(x + 2) * F]
            t = jnp.dot(sx, wpix_ref[...], preferred_element_type=jnp.float32)
            ix3_ref[x, :, :] = (t + diff_ref[:, x:x + 1]) * i1

    return pl.pallas_call(
        body,
        grid=(grid,),
        in_specs=[
            pl.BlockSpec((Eb, C), lambda i: (i, 0)),
            pl.BlockSpec((Eb, B), lambda i: (i + OFFB, 0)),
            pl.BlockSpec((Eb, 3), lambda i: (i + OFFB, 0)),
            pl.BlockSpec((F, F * B), lambda i: (0, 0)),
            pl.BlockSpec((F, F), lambda i: (0, 0)),
            pl.BlockSpec((F, F), lambda i: (0, 0)),
        ],
        out_specs=[
            pl.BlockSpec((Eb, F), lambda i: (i, 0)),
            pl.BlockSpec((3, Eb, F), lambda i: (0, i, 0)),
        ],
        out_shape=[
            jax.ShapeDtypeStruct((EH, F), jnp.float32),
            jax.ShapeDtypeStruct((3, EH, F), jnp.float32),
        ],
    )


def _make_scatter(N, F, EOFF, EH):
    ET = EH // 16
    BE = 96
    NB = ET // BE
    BT = ET - NB * BE
    NP = 80
    NPc = N // NP
    mesh = plsc.VectorSubcoreMesh(core_axis_name="c", subcore_axis_name="s")

    @functools.partial(
        pl.kernel,
        out_type=[
            jax.ShapeDtypeStruct((N, F), jnp.float32),
            jax.ShapeDtypeStruct((N, 3 * F), jnp.float32),
        ],
        mesh=mesh,
        scratch_types=[
            pltpu.VMEM((BE,), jnp.int32),
            pltpu.VMEM((BE,), jnp.int32),
            pltpu.VMEM((max(BT, 8),), jnp.int32),
            pltpu.VMEM((BE, F), jnp.float32),
            pltpu.VMEM((BE, F), jnp.float32),
            pltpu.VMEM((max(BT, 8), F), jnp.float32),
            pltpu.VMEM((NP, F), jnp.float32),
            pltpu.VMEM((NP, F), jnp.float32),
            pltpu.VMEM_SHARED((N, F), jnp.float32),
            pltpu.SemaphoreType.DMA,
            pltpu.SemaphoreType.DMA,
            pltpu.SemaphoreType.DMA,
            pltpu.SemaphoreType.DMA,
        ],
    )
    def scatter_k(i1_hbm, ix3_hbm, pairi_hbm, zeros_hbm, out1_hbm, out3_hbm,
                  idx0, idx1, idxt, r0b, r1b, rtb, zbuf, wbuf, acc_sh,
                  sI0, sR0, sI1, sR1):
        cid = lax.axis_index("c")
        sid = lax.axis_index("s")
        pltpu.sync_copy(zeros_hbm, zbuf)

        npieces = (NPc - sid + 15) // 16

        def run_pass(src_at, dst_at):
            def zero_piece(k, carry):
                r0 = (sid + 16 * k) * NP
                pltpu.sync_copy(zbuf, acc_sh.at[pl.ds(r0, NP)])
                return carry

            lax.fori_loop(0, npieces, zero_piece, 0)
            plsc.subcore_barrier()

            def fire(b, idx_v, rows_v, sI, sR):
                base = sid * ET + b * BE
                pltpu.async_copy(
                    pairi_hbm.at[pl.ds(EOFF + base, BE)], idx_v, sI)
                pltpu.async_copy(src_at(base, BE), rows_v, sR)

            def finish(b, idx_v, rows_v, sI, sR):
                base = sid * ET + b * BE
                pltpu.make_async_copy(
                    pairi_hbm.at[pl.ds(EOFF + base, BE)], idx_v, sI).wait()
                pltpu.make_async_copy(src_at(base, BE), rows_v, sR).wait()
                pltpu.sync_copy(rows_v, acc_sh.at[idx_v], add=True)

            fire(0, idx0, r0b, sI0, sR0)
            L = (NB - 1) // 2

            def blk(b2, carry):
                b0 = 2 * b2
                fire(b0 + 1, idx1, r1b, sI1, sR1)
                finish(b0, idx0, r0b, sI0, sR0)
                fire(b0 + 2, idx0, r0b, sI0, sR0)
                finish(b0 + 1, idx1, r1b, sI1, sR1)
                return carry

            lax.fori_loop(0, L, blk, 0)
            if NB % 2 == 1:
                finish(2 * L, idx0, r0b, sI0, sR0)
            else:
                fire(2 * L + 1, idx1, r1b, sI1, sR1)
                finish(2 * L, idx0, r0b, sI0, sR0)
                finish(2 * L + 1, idx1, r1b, sI1, sR1)
            if BT > 0:
                tbase = sid * ET + NB * BE
                pltpu.sync_copy(pairi_hbm.at[pl.ds(EOFF + tbase, BT)], idxt)
                pltpu.sync_copy(src_at(tbase, BT), rtb)
                pltpu.sync_copy(rtb, acc_sh.at[idxt], add=True)
            plsc.subcore_barrier()

            def write_piece(k, carry):
                r0 = (sid + 16 * k) * NP
                pltpu.sync_copy(acc_sh.at[pl.ds(r0, NP)], wbuf)
                pltpu.sync_copy(wbuf, dst_at(r0))
                return carry

            lax.fori_loop(0, npieces, write_piece, 0)

        @pl.when(cid == 0)
        def _():
            run_pass(lambda b, n: i1_hbm.at[pl.ds(b, n)],
                     lambda r: out1_hbm.at[pl.ds(r, NP)])
            run_pass(lambda b, n: ix3_hbm.at[0, pl.ds(b, n), :],
                     lambda r: out3_hbm.at[pl.ds(r, NP), pl.ds(0, F)])

        @pl.when(cid == 1)
        def _():
            run_pass(lambda b, n: ix3_hbm.at[1, pl.ds(b, n), :],
                     lambda r: out3_hbm.at[pl.ds(r, NP), pl.ds(F, F)])
            run_pass(lambda b, n: ix3_hbm.at[2, pl.ds(b, n), :],
                     lambda r: out3_hbm.at[pl.ds(r, NP), pl.ds(2 * F, F)])

    return scatter_k


def _make_node(N, F):
    Nb = 2000
    grid = N // Nb

    def body(a1_ref, a3_ref, wpp_ref, bpp_ref, weq_ref, wout_ref, bout_ref,
             p1t1_ref, p3t1_ref):
        p1n = jnp.tanh(
            jnp.dot(a1_ref[...], wpp_ref[...],
                    preferred_element_type=jnp.float32) + bpp_ref[...])
        p1t1_ref[:, 0, :] = jnp.dot(
            p1n, wout_ref[...], preferred_element_type=jnp.float32) + bout_ref[...]
        p3n = [
            jnp.dot(a3_ref[:, x * F:(x + 1) * F], weq_ref[...],
                    preferred_element_type=jnp.float32) for x in range(3)
        ]
        dot = p3n[0] * p3n[0] + p3n[1] * p3n[1] + p3n[2] * p3n[2]
        scale = jnp.dot(
            dot, wout_ref[...], preferred_element_type=jnp.float32) + bout_ref[...]
        for x in range(3):
            p3t1_ref[:, x, :] = p3n[x] * scale

    return pl.pallas_call(
        body,
        grid=(grid,),
        in_specs=[
            pl.BlockSpec((Nb, F), lambda i: (i, 0)),
            pl.BlockSpec((Nb, 3 * F), lambda i: (i, 0)),
            pl.BlockSpec((F, F), lambda i: (0, 0)),
            pl.BlockSpec((1, F), lambda i: (0, 0)),
            pl.BlockSpec((F, F), lambda i: (0, 0)),
            pl.BlockSpec((F, F), lambda i: (0, 0)),
            pl.BlockSpec((1, F), lambda i: (0, 0)),
        ],
        out_specs=[
            pl.BlockSpec((Nb, 1, F), lambda i: (i, 0, 0)),
            pl.BlockSpec((Nb, 3, F), lambda i: (i, 0, 0)),
        ],
        out_shape=[
            jax.ShapeDtypeStruct((N, 1, F), jnp.float32),
            jax.ShapeDtypeStruct((N, 3, F), jnp.float32),
        ],
    )


def kernel(p1, p3, pair_i, pair_j, basis, diff, W_pp, b_pp, W_pi, W_ii,
           W_eq_pp, W_pix, W_out, b_out):
    N, _, F = p1.shape
    E = pair_i.shape[0]
    B = basis.shape[1]

    cat = jnp.concatenate([p1.reshape(N, F), p3.reshape(N, 3 * F)], axis=1)
    W_pi_perm = W_pi.reshape(F, F, B).transpose(0, 2, 1).reshape(F, F * B)
    zeros = jnp.zeros((80, F), jnp.float32)

    s = _make_gather(N, 4 * F, 0, E)(cat, pair_i, pair_j)
    i1f, ix3 = _make_edge(E, F, B, 0)(s, basis, diff, W_pi_perm, W_ii, W_pix)
    acc1, acc3 = _make_scatter(N, F, 0, E)(i1f, ix3, pair_i, zeros)
    p1t1, p3t1 = _make_node(N, F)(
        acc1, acc3, W_pp, b_pp.reshape(1, F), W_eq_pp, W_out,
        b_out.reshape(1, F))
    return (p1t1, p3t1, i1f.reshape(E, 1, F), ix3.transpose(1, 0, 2))


# speedup vs baseline: 1.1071x; 1.0808x over previous
import functools

import jax
import jax.numpy as jnp
from jax import lax
from jax.experimental import pallas as pl
from jax.experimental.pallas import tpu as pltpu
from jax.experimental.pallas import tpu_sc as plsc


def _make_gather(N, C, EOFF, EH):
    NW = 32
    EW = EH // NW
    BE = 40
    NB = EW // BE
    mesh = plsc.VectorSubcoreMesh(core_axis_name="c", subcore_axis_name="s")

    @functools.partial(
        pl.kernel,
        out_type=jax.ShapeDtypeStruct((EH, C), jnp.float32),
        mesh=mesh,
        scratch_types=[
            pltpu.VMEM((EW,), jnp.int32),
            pltpu.VMEM((EW,), jnp.int32),
            pltpu.VMEM((BE, C), jnp.float32),
            pltpu.VMEM((BE, C), jnp.float32),
            pltpu.VMEM((BE, C), jnp.float32),
            pltpu.VMEM((BE, C), jnp.float32),
            pltpu.SemaphoreType.DMA,
            pltpu.SemaphoreType.DMA,
            pltpu.SemaphoreType.DMA,
            pltpu.SemaphoreType.DMA,
        ],
    )
    def gather_k(cat_hbm, pi_hbm, pj_hbm, s_hbm, idx_ia, idx_ja,
                 ri0, rj0, ri1, rj1, si0, sj0, si1, sj1):
        cid = lax.axis_index("c")
        sid = lax.axis_index("s")
        wid = sid * 2 + cid
        wbase = wid * EW
        pltpu.sync_copy(pi_hbm.at[pl.ds(EOFF + wbase, EW)], idx_ia)
        pltpu.sync_copy(pj_hbm.at[pl.ds(EOFF + wbase, EW)], idx_ja)

        def fire(b, ri, rj, si, sj):
            pltpu.async_copy(cat_hbm.at[idx_ia.at[pl.ds(b * BE, BE)]], ri, si)
            pltpu.async_copy(cat_hbm.at[idx_ja.at[pl.ds(b * BE, BE)]], rj, sj)

        def finish(b, ri, rj, si, sj):
            pltpu.make_async_copy(
                cat_hbm.at[idx_ia.at[pl.ds(b * BE, BE)]], ri, si).wait()
            pltpu.make_async_copy(
                cat_hbm.at[idx_ja.at[pl.ds(b * BE, BE)]], rj, sj).wait()

            def add_row(e, c2):
                for g in range(C // 16):
                    sl = pl.ds(g * 16, 16)
                    ri[e, sl] = ri[e, sl] + rj[e, sl]
                return c2

            lax.fori_loop(0, BE, add_row, 0)
            pltpu.sync_copy(ri, s_hbm.at[pl.ds(wbase + b * BE, BE)])

        fire(0, ri0, rj0, si0, sj0)
        L = (NB - 1) // 2

        def body(b2, carry):
            b0 = 2 * b2
            fire(b0 + 1, ri1, rj1, si1, sj1)
            finish(b0, ri0, rj0, si0, sj0)
            fire(b0 + 2, ri0, rj0, si0, sj0)
            finish(b0 + 1, ri1, rj1, si1, sj1)
            return carry

        lax.fori_loop(0, L, body, 0)
        if NB % 2 == 1:
            finish(2 * L, ri0, rj0, si0, sj0)
        else:
            fire(2 * L + 1, ri1, rj1, si1, sj1)
            finish(2 * L, ri0, rj0, si0, sj0)
            finish(2 * L + 1, ri1, rj1, si1, sj1)

    return gather_k


def _make_edge(EH, F, B, OFFB):
    Eb = 1280
    grid = EH // Eb
    C = 4 * F

    def body(s_ref, basis_ref, diff_ref, wpi_ref, wii_ref, wpix_ref,
             i1_ref, ix3_ref):
        s1 = s_ref[:, :F]
        inter = jnp.tanh(
            jnp.dot(s1, wpi_ref[...], preferred_element_type=jnp.float32))
        u = inter[:, 0:F] * basis_ref[:, 0:1]
        for b in range(1, B):
            u = u + inter[:, b * F:(b + 1) * F] * basis_ref[:, b:b + 1]
        i1 = jnp.tanh(
            jnp.dot(u, wii_ref[...], preferred_element_type=jnp.float32))
        i1_ref[...] = i1
        for x in range(3):
            sx = s_ref[:, (x + 1) * F:(x + 2) * F]
            t = jnp.dot(sx, wpix_ref[...], preferred_element_type=jnp.float32)
            ix3_ref[x, :, :] = (t + diff_ref[:, x:x + 1]) * i1

    return pl.pallas_call(
        body,
        grid=(grid,),
        in_specs=[
            pl.BlockSpec((Eb, C), lambda i: (i, 0)),
            pl.BlockSpec((Eb, B), lambda i: (i + OFFB, 0)),
            pl.BlockSpec((Eb, 3), lambda i: (i + OFFB, 0)),
            pl.BlockSpec((F, F * B), lambda i: (0, 0)),
            pl.BlockSpec((F, F), lambda i: (0, 0)),
            pl.BlockSpec((F, F), lambda i: (0, 0)),
        ],
        out_specs=[
            pl.BlockSpec((Eb, F), lambda i: (i, 0)),
            pl.BlockSpec((3, Eb, F), lambda i: (0, i, 0)),
        ],
        out_shape=[
            jax.ShapeDtypeStruct((EH, F), jnp.float32),
            jax.ShapeDtypeStruct((3, EH, F), jnp.float32),
        ],
    )


def _make_scatter(N, F, EOFF, EH):
    ET = EH // 16
    BE = 96
    NB = ET // BE
    BT = ET - NB * BE
    NP = 80
    NPc = N // NP
    mesh = plsc.VectorSubcoreMesh(core_axis_name="c", subcore_axis_name="s")

    @functools.partial(
        pl.kernel,
        out_type=[
            jax.ShapeDtypeStruct((N, F), jnp.float32),
            jax.ShapeDtypeStruct((N, 3 * F), jnp.float32),
        ],
        mesh=mesh,
        scratch_types=[
            pltpu.VMEM((BE,), jnp.int32),
            pltpu.VMEM((BE,), jnp.int32),
            pltpu.VMEM((max(BT, 8),), jnp.int32),
            pltpu.VMEM((BE, F), jnp.float32),
            pltpu.VMEM((BE, F), jnp.float32),
            pltpu.VMEM((max(BT, 8), F), jnp.float32),
            pltpu.VMEM((NP, F), jnp.float32),
            pltpu.VMEM((NP, F), jnp.float32),
            pltpu.VMEM_SHARED((N, F), jnp.float32),
            pltpu.SemaphoreType.DMA,
            pltpu.SemaphoreType.DMA,
            pltpu.SemaphoreType.DMA,
            pltpu.SemaphoreType.DMA,
        ],
    )
    def scatter_k(i1_hbm, ix3_hbm, pairi_hbm, zeros_hbm, out1_hbm, out3_hbm,
                  idx0, idx1, idxt, r0b, r1b, rtb, zbuf, wbuf, acc_sh,
                  sI0, sR0, sI1, sR1):
        cid = lax.axis_index("c")
        sid = lax.axis_index("s")
        pltpu.sync_copy(zeros_hbm, zbuf)

        npieces = (NPc - sid + 15) // 16

        def run_pass(src_at, dst_at):
            def zero_piece(k, carry):
                r0 = (sid + 16 * k) * NP
                pltpu.sync_copy(zbuf, acc_sh.at[pl.ds(r0, NP)])
                return carry

            lax.fori_loop(0, npieces, zero_piece, 0)
            plsc.subcore_barrier()

            def fire(b, idx_v, rows_v, sI, sR):
                base = sid * ET + b * BE
                pltpu.async_copy(
                    pairi_hbm.at[pl.ds(EOFF + base, BE)], idx_v, sI)
                pltpu.async_copy(src_at(base, BE), rows_v, sR)

            def finish(b, idx_v, rows_v, sI, sR):
                base = sid * ET + b * BE
                pltpu.make_async_copy(
                    pairi_hbm.at[pl.ds(EOFF + base, BE)], idx_v, sI).wait()
                pltpu.make_async_copy(src_at(base, BE), rows_v, sR).wait()
                pltpu.sync_copy(rows_v, acc_sh.at[idx_v], add=True)

            fire(0, idx0, r0b, sI0, sR0)
            L = (NB - 1) // 2

            def blk(b2, carry):
                b0 = 2 * b2
                fire(b0 + 1, idx1, r1b, sI1, sR1)
                finish(b0, idx0, r0b, sI0, sR0)
                fire(b0 + 2, idx0, r0b, sI0, sR0)
                finish(b0 + 1, idx1, r1b, sI1, sR1)
                return carry

            lax.fori_loop(0, L, blk, 0)
            if NB % 2 == 1:
                finish(2 * L, idx0, r0b, sI0, sR0)
            else:
                fire(2 * L + 1, idx1, r1b, sI1, sR1)
                finish(2 * L, idx0, r0b, sI0, sR0)
                finish(2 * L + 1, idx1, r1b, sI1, sR1)
            if BT > 0:
                tbase = sid * ET + NB * BE
                pltpu.sync_copy(pairi_hbm.at[pl.ds(EOFF + tbase, BT)], idxt)
                pltpu.sync_copy(src_at(tbase, BT), rtb)
                pltpu.sync_copy(rtb, acc_sh.at[idxt], add=True)
            plsc.subcore_barrier()

            def write_piece(k, carry):
                r0 = (sid + 16 * k) * NP
                pltpu.sync_copy(acc_sh.at[pl.ds(r0, NP)], wbuf)
                pltpu.sync_copy(wbuf, dst_at(r0))
                return carry

            lax.fori_loop(0, npieces, write_piece, 0)

        @pl.when(cid == 0)
        def _():
            run_pass(lambda b, n: i1_hbm.at[pl.ds(b, n)],
                     lambda r: out1_hbm.at[pl.ds(r, NP)])
            run_pass(lambda b, n: ix3_hbm.at[0, pl.ds(b, n), :],
                     lambda r: out3_hbm.at[pl.ds(r, NP), pl.ds(0, F)])

        @pl.when(cid == 1)
        def _():
            run_pass(lambda b, n: ix3_hbm.at[1, pl.ds(b, n), :],
                     lambda r: out3_hbm.at[pl.ds(r, NP), pl.ds(F, F)])
            run_pass(lambda b, n: ix3_hbm.at[2, pl.ds(b, n), :],
                     lambda r: out3_hbm.at[pl.ds(r, NP), pl.ds(2 * F, F)])

    return scatter_k


def _make_node(N, F):
    Nb = 2000
    grid = N // Nb

    def body(a1_ref, a3_ref, wpp_ref, bpp_ref, weq_ref, wout_ref, bout_ref,
             p1t1_ref, p3t1_ref):
        p1n = jnp.tanh(
            jnp.dot(a1_ref[...], wpp_ref[...],
                    preferred_element_type=jnp.float32) + bpp_ref[...])
        p1t1_ref[:, 0, :] = jnp.dot(
            p1n, wout_ref[...], preferred_element_type=jnp.float32) + bout_ref[...]
        p3n = [
            jnp.dot(a3_ref[:, x * F:(x + 1) * F], weq_ref[...],
                    preferred_element_type=jnp.float32) for x in range(3)
        ]
        dot = p3n[0] * p3n[0] + p3n[1] * p3n[1] + p3n[2] * p3n[2]
        scale = jnp.dot(
            dot, wout_ref[...], preferred_element_type=jnp.float32) + bout_ref[...]
        for x in range(3):
            p3t1_ref[:, x, :] = p3n[x] * scale

    return pl.pallas_call(
        body,
        grid=(grid,),
        in_specs=[
            pl.BlockSpec((Nb, F), lambda i: (i, 0)),
            pl.BlockSpec((Nb, 3 * F), lambda i: (i, 0)),
            pl.BlockSpec((F, F), lambda i: (0, 0)),
            pl.BlockSpec((1, F), lambda i: (0, 0)),
            pl.BlockSpec((F, F), lambda i: (0, 0)),
            pl.BlockSpec((F, F), lambda i: (0, 0)),
            pl.BlockSpec((1, F), lambda i: (0, 0)),
        ],
        out_specs=[
            pl.BlockSpec((Nb, 1, F), lambda i: (i, 0, 0)),
            pl.BlockSpec((Nb, 3, F), lambda i: (i, 0, 0)),
        ],
        out_shape=[
            jax.ShapeDtypeStruct((N, 1, F), jnp.float32),
            jax.ShapeDtypeStruct((N, 3, F), jnp.float32),
        ],
    )


def kernel(p1, p3, pair_i, pair_j, basis, diff, W_pp, b_pp, W_pi, W_ii,
           W_eq_pp, W_pix, W_out, b_out):
    N, _, F = p1.shape
    E = pair_i.shape[0]
    B = basis.shape[1]

    cat = jnp.concatenate([p1.reshape(N, F), p3.reshape(N, 3 * F)], axis=1)
    W_pi_perm = W_pi.reshape(F, F, B).transpose(0, 2, 1).reshape(F, F * B)
    zeros = jnp.zeros((80, F), jnp.float32)

    s = _make_gather(N, 4 * F, 0, E)(cat, pair_i, pair_j)
    i1f, ix3 = _make_edge(E, F, B, 0)(s, basis, diff, W_pi_perm, W_ii, W_pix)
    acc1, acc3 = _make_scatter(N, F, 0, E)(i1f, ix3, pair_i, zeros)
    p1t1, p3t1 = _make_node(N, F)(
        acc1, acc3, W_pp, b_pp.reshape(1, F), W_eq_pp, W_out,
        b_out.reshape(1, F))
    return (p1t1, p3t1, i1f.reshape(E, 1, F), ix3.transpose(1, 0, 2))


# speedup vs baseline: 1.1268x; 1.0178x over previous
import functools

import jax
import jax.numpy as jnp
from jax import lax
from jax.experimental import pallas as pl
from jax.experimental.pallas import tpu as pltpu
from jax.experimental.pallas import tpu_sc as plsc


def _make_gather(N, C, EOFF, EH):
    NW = 32
    EW = EH // NW
    BE = 40
    NB = EW // BE
    mesh = plsc.VectorSubcoreMesh(core_axis_name="c", subcore_axis_name="s")

    @functools.partial(
        pl.kernel,
        out_type=jax.ShapeDtypeStruct((EH, C), jnp.float32),
        mesh=mesh,
        scratch_types=[
            pltpu.VMEM((EW,), jnp.int32),
            pltpu.VMEM((EW,), jnp.int32),
            pltpu.VMEM((BE, C), jnp.float32),
            pltpu.VMEM((BE, C), jnp.float32),
            pltpu.VMEM((BE, C), jnp.float32),
            pltpu.VMEM((BE, C), jnp.float32),
            pltpu.SemaphoreType.DMA,
            pltpu.SemaphoreType.DMA,
            pltpu.SemaphoreType.DMA,
            pltpu.SemaphoreType.DMA,
        ],
    )
    def gather_k(cat_hbm, pi_hbm, pj_hbm, s_hbm, idx_ia, idx_ja,
                 ri0, rj0, ri1, rj1, si0, sj0, si1, sj1):
        cid = lax.axis_index("c")
        sid = lax.axis_index("s")
        wid = sid * 2 + cid
        wbase = wid * EW
        pltpu.sync_copy(pi_hbm.at[pl.ds(EOFF + wbase, EW)], idx_ia)
        pltpu.sync_copy(pj_hbm.at[pl.ds(EOFF + wbase, EW)], idx_ja)

        def fire(b, ri, rj, si, sj):
            pltpu.async_copy(cat_hbm.at[idx_ia.at[pl.ds(b * BE, BE)]], ri, si)
            pltpu.async_copy(cat_hbm.at[idx_ja.at[pl.ds(b * BE, BE)]], rj, sj)

        def finish(b, ri, rj, si, sj):
            pltpu.make_async_copy(
                cat_hbm.at[idx_ia.at[pl.ds(b * BE, BE)]], ri, si).wait()
            pltpu.make_async_copy(
                cat_hbm.at[idx_ja.at[pl.ds(b * BE, BE)]], rj, sj).wait()

            def add_row(e, c2):
                for g in range(C // 16):
                    sl = pl.ds(g * 16, 16)
                    ri[e, sl] = ri[e, sl] + rj[e, sl]
                return c2

            lax.fori_loop(0, BE, add_row, 0)
            pltpu.sync_copy(ri, s_hbm.at[pl.ds(wbase + b * BE, BE)])

        fire(0, ri0, rj0, si0, sj0)
        L = (NB - 1) // 2

        def body(b2, carry):
            b0 = 2 * b2
            fire(b0 + 1, ri1, rj1, si1, sj1)
            finish(b0, ri0, rj0, si0, sj0)
            fire(b0 + 2, ri0, rj0, si0, sj0)
            finish(b0 + 1, ri1, rj1, si1, sj1)
            return carry

        lax.fori_loop(0, L, body, 0)
        if NB % 2 == 1:
            finish(2 * L, ri0, rj0, si0, sj0)
        else:
            fire(2 * L + 1, ri1, rj1, si1, sj1)
            finish(2 * L, ri0, rj0, si0, sj0)
            finish(2 * L + 1, ri1, rj1, si1, sj1)

    return gather_k


def _make_edge(EH, F, B, OFFB):
    Eb = 2000
    grid = EH // Eb
    C = 4 * F

    def body(s_ref, basis_ref, diff_ref, wpi_ref, wii_ref, wpix_ref,
             i1_ref, ix3_ref):
        s1 = s_ref[:, :F]
        inter = jnp.tanh(
            jnp.dot(s1, wpi_ref[...], preferred_element_type=jnp.float32))
        u = inter[:, 0:F] * basis_ref[:, 0:1]
        for b in range(1, B):
            u = u + inter[:, b * F:(b + 1) * F] * basis_ref[:, b:b + 1]
        i1 = jnp.tanh(
            jnp.dot(u, wii_ref[...], preferred_element_type=jnp.float32))
        i1_ref[...] = i1
        for x in range(3):
            sx = s_ref[:, (x + 1) * F:(x + 2) * F]
            t = jnp.dot(sx, wpix_ref[...], preferred_element_type=jnp.float32)
            ix3_ref[x, :, :] = (t + diff_ref[:, x:x + 1]) * i1

    return pl.pallas_call(
        body,
        grid=(grid,),
        in_specs=[
            pl.BlockSpec((Eb, C), lambda i: (i, 0)),
            pl.BlockSpec((Eb, B), lambda i: (i + OFFB, 0)),
            pl.BlockSpec((Eb, 3), lambda i: (i + OFFB, 0)),
            pl.BlockSpec((F, F * B), lambda i: (0, 0)),
            pl.BlockSpec((F, F), lambda i: (0, 0)),
            pl.BlockSpec((F, F), lambda i: (0, 0)),
        ],
        out_specs=[
            pl.BlockSpec((Eb, F), lambda i: (i, 0)),
            pl.BlockSpec((3, Eb, F), lambda i: (0, i, 0)),
        ],
        out_shape=[
            jax.ShapeDtypeStruct((EH, F), jnp.float32),
            jax.ShapeDtypeStruct((3, EH, F), jnp.float32),
        ],
    )


def _make_scatter(N, F, EOFF, EH):
    ET = EH // 16
    BE = 96
    NB = ET // BE
    BT = ET - NB * BE
    NP = 80
    NPc = N // NP
    mesh = plsc.VectorSubcoreMesh(core_axis_name="c", subcore_axis_name="s")

    @functools.partial(
        pl.kernel,
        out_type=[
            jax.ShapeDtypeStruct((N, F), jnp.float32),
            jax.ShapeDtypeStruct((N, 3 * F), jnp.float32),
        ],
        mesh=mesh,
        scratch_types=[
            pltpu.VMEM((BE,), jnp.int32),
            pltpu.VMEM((BE,), jnp.int32),
            pltpu.VMEM((max(BT, 8),), jnp.int32),
            pltpu.VMEM((BE, F), jnp.float32),
            pltpu.VMEM((BE, F), jnp.float32),
            pltpu.VMEM((max(BT, 8), F), jnp.float32),
            pltpu.VMEM((NP, F), jnp.float32),
            pltpu.VMEM((NP, F), jnp.float32),
            pltpu.VMEM_SHARED((N, F), jnp.float32),
            pltpu.SemaphoreType.DMA,
            pltpu.SemaphoreType.DMA,
            pltpu.SemaphoreType.DMA,
            pltpu.SemaphoreType.DMA,
        ],
    )
    def scatter_k(i1_hbm, ix3_hbm, pairi_hbm, zeros_hbm, out1_hbm, out3_hbm,
                  idx0, idx1, idxt, r0b, r1b, rtb, zbuf, wbuf, acc_sh,
                  sI0, sR0, sI1, sR1):
        cid = lax.axis_index("c")
        sid = lax.axis_index("s")
        pltpu.sync_copy(zeros_hbm, zbuf)

        npieces = (NPc - sid + 15) // 16

        def run_pass(src_at, dst_at):
            def zero_piece(k, carry):
                r0 = (sid + 16 * k) * NP
                pltpu.sync_copy(zbuf, acc_sh.at[pl.ds(r0, NP)])
                return carry

            lax.fori_loop(0, npieces, zero_piece, 0)
            plsc.subcore_barrier()

            def fire(b, idx_v, rows_v, sI, sR):
                base = sid * ET + b * BE
                pltpu.async_copy(
                    pairi_hbm.at[pl.ds(EOFF + base, BE)], idx_v, sI)
                pltpu.async_copy(src_at(base, BE), rows_v, sR)

            def finish(b, idx_v, rows_v, sI, sR):
                base = sid * ET + b * BE
                pltpu.make_async_copy(
                    pairi_hbm.at[pl.ds(EOFF + base, BE)], idx_v, sI).wait()
                pltpu.make_async_copy(src_at(base, BE), rows_v, sR).wait()
                pltpu.sync_copy(rows_v, acc_sh.at[idx_v], add=True)

            fire(0, idx0, r0b, sI0, sR0)
            L = (NB - 1) // 2

            def blk(b2, carry):
                b0 = 2 * b2
                fire(b0 + 1, idx1, r1b, sI1, sR1)
                finish(b0, idx0, r0b, sI0, sR0)
                fire(b0 + 2, idx0, r0b, sI0, sR0)
                finish(b0 + 1, idx1, r1b, sI1, sR1)
                return carry

            lax.fori_loop(0, L, blk, 0)
            if NB % 2 == 1:
                finish(2 * L, idx0, r0b, sI0, sR0)
            else:
                fire(2 * L + 1, idx1, r1b, sI1, sR1)
                finish(2 * L, idx0, r0b, sI0, sR0)
                finish(2 * L + 1, idx1, r1b, sI1, sR1)
            if BT > 0:
                tbase = sid * ET + NB * BE
                pltpu.sync_copy(pairi_hbm.at[pl.ds(EOFF + tbase, BT)], idxt)
                pltpu.sync_copy(src_at(tbase, BT), rtb)
                pltpu.sync_copy(rtb, acc_sh.at[idxt], add=True)
            plsc.subcore_barrier()

            def write_piece(k, carry):
                r0 = (sid + 16 * k) * NP
                pltpu.sync_copy(acc_sh.at[pl.ds(r0, NP)], wbuf)
                pltpu.sync_copy(wbuf, dst_at(r0))
                return carry

            lax.fori_loop(0, npieces, write_piece, 0)

        @pl.when(cid == 0)
        def _():
            run_pass(lambda b, n: i1_hbm.at[pl.ds(b, n)],
                     lambda r: out1_hbm.at[pl.ds(r, NP)])
            run_pass(lambda b, n: ix3_hbm.at[0, pl.ds(b, n), :],
                     lambda r: out3_hbm.at[pl.ds(r, NP), pl.ds(0, F)])

        @pl.when(cid == 1)
        def _():
            run_pass(lambda b, n: ix3_hbm.at[1, pl.ds(b, n), :],
                     lambda r: out3_hbm.at[pl.ds(r, NP), pl.ds(F, F)])
            run_pass(lambda b, n: ix3_hbm.at[2, pl.ds(b, n), :],
                     lambda r: out3_hbm.at[pl.ds(r, NP), pl.ds(2 * F, F)])

    return scatter_k


def _make_node(N, F):
    Nb = 2000
    grid = N // Nb

    def body(a1_ref, a3_ref, wpp_ref, bpp_ref, weq_ref, wout_ref, bout_ref,
             p1t1_ref, p3t1_ref):
        p1n = jnp.tanh(
            jnp.dot(a1_ref[...], wpp_ref[...],
                    preferred_element_type=jnp.float32) + bpp_ref[...])
        p1t1_ref[:, 0, :] = jnp.dot(
            p1n, wout_ref[...], preferred_element_type=jnp.float32) + bout_ref[...]
        p3n = [
            jnp.dot(a3_ref[:, x * F:(x + 1) * F], weq_ref[...],
                    preferred_element_type=jnp.float32) for x in range(3)
        ]
        dot = p3n[0] * p3n[0] + p3n[1] * p3n[1] + p3n[2] * p3n[2]
        scale = jnp.dot(
            dot, wout_ref[...], preferred_element_type=jnp.float32) + bout_ref[...]
        for x in range(3):
            p3t1_ref[:, x, :] = p3n[x] * scale

    return pl.pallas_call(
        body,
        grid=(grid,),
        in_specs=[
            pl.BlockSpec((Nb, F), lambda i: (i, 0)),
            pl.BlockSpec((Nb, 3 * F), lambda i: (i, 0)),
            pl.BlockSpec((F, F), lambda i: (0, 0)),
            pl.BlockSpec((1, F), lambda i: (0, 0)),
            pl.BlockSpec((F, F), lambda i: (0, 0)),
            pl.BlockSpec((F, F), lambda i: (0, 0)),
            pl.BlockSpec((1, F), lambda i: (0, 0)),
        ],
        out_specs=[
            pl.BlockSpec((Nb, 1, F), lambda i: (i, 0, 0)),
            pl.BlockSpec((Nb, 3, F), lambda i: (i, 0, 0)),
        ],
        out_shape=[
            jax.ShapeDtypeStruct((N, 1, F), jnp.float32),
            jax.ShapeDtypeStruct((N, 3, F), jnp.float32),
        ],
    )


def kernel(p1, p3, pair_i, pair_j, basis, diff, W_pp, b_pp, W_pi, W_ii,
           W_eq_pp, W_pix, W_out, b_out):
    N, _, F = p1.shape
    E = pair_i.shape[0]
    B = basis.shape[1]

    cat = jnp.concatenate([p1.reshape(N, F), p3.reshape(N, 3 * F)], axis=1)
    W_pi_perm = W_pi.reshape(F, F, B).transpose(0, 2, 1).reshape(F, F * B)
    zeros = jnp.zeros((80, F), jnp.float32)

    s = _make_gather(N, 4 * F, 0, E)(cat, pair_i, pair_j)
    i1f, ix3 = _make_edge(E, F, B, 0)(s, basis, diff, W_pi_perm, W_ii, W_pix)
    acc1, acc3 = _make_scatter(N, F, 0, E)(i1f, ix3, pair_i, zeros)
    p1t1, p3t1 = _make_node(N, F)(
        acc1, acc3, W_pp, b_pp.reshape(1, F), W_eq_pp, W_out,
        b_out.reshape(1, F))
    return (p1t1, p3t1, i1f.reshape(E, 1, F), ix3.transpose(1, 0, 2))


# speedup vs baseline: 1.1328x; 1.0053x over previous
import functools

import jax
import jax.numpy as jnp
from jax import lax
from jax.experimental import pallas as pl
from jax.experimental.pallas import tpu as pltpu
from jax.experimental.pallas import tpu_sc as plsc


def _make_gather(N, C, EOFF, EH):
    NW = 32
    EW = EH // NW
    BE = 40
    NB = EW // BE
    mesh = plsc.VectorSubcoreMesh(core_axis_name="c", subcore_axis_name="s")

    @functools.partial(
        pl.kernel,
        out_type=jax.ShapeDtypeStruct((EH, C), jnp.float32),
        mesh=mesh,
        scratch_types=[
            pltpu.VMEM((EW,), jnp.int32),
            pltpu.VMEM((EW,), jnp.int32),
            pltpu.VMEM((BE, C), jnp.float32),
            pltpu.VMEM((BE, C), jnp.float32),
            pltpu.VMEM((BE, C), jnp.float32),
            pltpu.VMEM((BE, C), jnp.float32),
            pltpu.SemaphoreType.DMA,
            pltpu.SemaphoreType.DMA,
            pltpu.SemaphoreType.DMA,
            pltpu.SemaphoreType.DMA,
        ],
    )
    def gather_k(cat_hbm, pi_hbm, pj_hbm, s_hbm, idx_ia, idx_ja,
                 ri0, rj0, ri1, rj1, si0, sj0, si1, sj1):
        cid = lax.axis_index("c")
        sid = lax.axis_index("s")
        wid = sid * 2 + cid
        wbase = wid * EW
        pltpu.sync_copy(pi_hbm.at[pl.ds(EOFF + wbase, EW)], idx_ia)
        pltpu.sync_copy(pj_hbm.at[pl.ds(EOFF + wbase, EW)], idx_ja)

        def fire(b, ri, rj, si, sj):
            pltpu.async_copy(cat_hbm.at[idx_ia.at[pl.ds(b * BE, BE)]], ri, si)
            pltpu.async_copy(cat_hbm.at[idx_ja.at[pl.ds(b * BE, BE)]], rj, sj)

        def finish(b, ri, rj, si, sj):
            pltpu.make_async_copy(
                cat_hbm.at[idx_ia.at[pl.ds(b * BE, BE)]], ri, si).wait()
            pltpu.make_async_copy(
                cat_hbm.at[idx_ja.at[pl.ds(b * BE, BE)]], rj, sj).wait()

            def add_row(e, c2):
                for g in range(C // 16):
                    sl = pl.ds(g * 16, 16)
                    ri[e, sl] = ri[e, sl] + rj[e, sl]
                return c2

            lax.fori_loop(0, BE, add_row, 0)
            pltpu.sync_copy(ri, s_hbm.at[pl.ds(wbase + b * BE, BE)])

        fire(0, ri0, rj0, si0, sj0)
        L = (NB - 1) // 2

        def body(b2, carry):
            b0 = 2 * b2
            fire(b0 + 1, ri1, rj1, si1, sj1)
            finish(b0, ri0, rj0, si0, sj0)
            fire(b0 + 2, ri0, rj0, si0, sj0)
            finish(b0 + 1, ri1, rj1, si1, sj1)
            return carry

        lax.fori_loop(0, L, body, 0)
        if NB % 2 == 1:
            finish(2 * L, ri0, rj0, si0, sj0)
        else:
            fire(2 * L + 1, ri1, rj1, si1, sj1)
            finish(2 * L, ri0, rj0, si0, sj0)
            finish(2 * L + 1, ri1, rj1, si1, sj1)

    return gather_k


def _make_edge(EH, F, B, OFFB):
    Eb = 3200
    grid = EH // Eb
    C = 4 * F

    def body(s_ref, basis_ref, diff_ref, wpi_ref, wii_ref, wpix_ref,
             i1_ref, ix3_ref):
        s1 = s_ref[:, :F]
        inter = jnp.tanh(
            jnp.dot(s1, wpi_ref[...], preferred_element_type=jnp.float32))
        u = inter[:, 0:F] * basis_ref[:, 0:1]
        for b in range(1, B):
            u = u + inter[:, b * F:(b + 1) * F] * basis_ref[:, b:b + 1]
        i1 = jnp.tanh(
            jnp.dot(u, wii_ref[...], preferred_element_type=jnp.float32))
        i1_ref[...] = i1
        for x in range(3):
            sx = s_ref[:, (x + 1) * F:(x + 2) * F]
            t = jnp.dot(sx, wpix_ref[...], preferred_element_type=jnp.float32)
            ix3_ref[x, :, :] = (t + diff_ref[:, x:x + 1]) * i1

    return pl.pallas_call(
        body,
        grid=(grid,),
        in_specs=[
            pl.BlockSpec((Eb, C), lambda i: (i, 0)),
            pl.BlockSpec((Eb, B), lambda i: (i + OFFB, 0)),
            pl.BlockSpec((Eb, 3), lambda i: (i + OFFB, 0)),
            pl.BlockSpec((F, F * B), lambda i: (0, 0)),
            pl.BlockSpec((F, F), lambda i: (0, 0)),
            pl.BlockSpec((F, F), lambda i: (0, 0)),
        ],
        out_specs=[
            pl.BlockSpec((Eb, F), lambda i: (i, 0)),
            pl.BlockSpec((3, Eb, F), lambda i: (0, i, 0)),
        ],
        out_shape=[
            jax.ShapeDtypeStruct((EH, F), jnp.float32),
            jax.ShapeDtypeStruct((3, EH, F), jnp.float32),
        ],
    )


def _make_scatter(N, F, EOFF, EH):
    ET = EH // 16
    BE = 96
    NB = ET // BE
    BT = ET - NB * BE
    NP = 80
    NPc = N // NP
    mesh = plsc.VectorSubcoreMesh(core_axis_name="c", subcore_axis_name="s")

    @functools.partial(
        pl.kernel,
        out_type=[
            jax.ShapeDtypeStruct((N, F), jnp.float32),
            jax.ShapeDtypeStruct((N, 3 * F), jnp.float32),
        ],
        mesh=mesh,
        scratch_types=[
            pltpu.VMEM((BE,), jnp.int32),
            pltpu.VMEM((BE,), jnp.int32),
            pltpu.VMEM((max(BT, 8),), jnp.int32),
            pltpu.VMEM((BE, F), jnp.float32),
            pltpu.VMEM((BE, F), jnp.float32),
            pltpu.VMEM((max(BT, 8), F), jnp.float32),
            pltpu.VMEM((NP, F), jnp.float32),
            pltpu.VMEM((NP, F), jnp.float32),
            pltpu.VMEM_SHARED((N, F), jnp.float32),
            pltpu.SemaphoreType.DMA,
            pltpu.SemaphoreType.DMA,
            pltpu.SemaphoreType.DMA,
            pltpu.SemaphoreType.DMA,
        ],
    )
    def scatter_k(i1_hbm, ix3_hbm, pairi_hbm, zeros_hbm, out1_hbm, out3_hbm,
                  idx0, idx1, idxt, r0b, r1b, rtb, zbuf, wbuf, acc_sh,
                  sI0, sR0, sI1, sR1):
        cid = lax.axis_index("c")
        sid = lax.axis_index("s")
        pltpu.sync_copy(zeros_hbm, zbuf)

        npieces = (NPc - sid + 15) // 16

        def run_pass(src_at, dst_at):
            def zero_piece(k, carry):
                r0 = (sid + 16 * k) * NP
                pltpu.sync_copy(zbuf, acc_sh.at[pl.ds(r0, NP)])
                return carry

            lax.fori_loop(0, npieces, zero_piece, 0)
            plsc.subcore_barrier()

            def fire(b, idx_v, rows_v, sI, sR):
                base = sid * ET + b * BE
                pltpu.async_copy(
                    pairi_hbm.at[pl.ds(EOFF + base, BE)], idx_v, sI)
                pltpu.async_copy(src_at(base, BE), rows_v, sR)

            def finish(b, idx_v, rows_v, sI, sR):
                base = sid * ET + b * BE
                pltpu.make_async_copy(
                    pairi_hbm.at[pl.ds(EOFF + base, BE)], idx_v, sI).wait()
                pltpu.make_async_copy(src_at(base, BE), rows_v, sR).wait()
                pltpu.sync_copy(rows_v, acc_sh.at[idx_v], add=True)

            fire(0, idx0, r0b, sI0, sR0)
            L = (NB - 1) // 2

            def blk(b2, carry):
                b0 = 2 * b2
                fire(b0 + 1, idx1, r1b, sI1, sR1)
                finish(b0, idx0, r0b, sI0, sR0)
                fire(b0 + 2, idx0, r0b, sI0, sR0)
                finish(b0 + 1, idx1, r1b, sI1, sR1)
                return carry

            lax.fori_loop(0, L, blk, 0)
            if NB % 2 == 1:
                finish(2 * L, idx0, r0b, sI0, sR0)
            else:
                fire(2 * L + 1, idx1, r1b, sI1, sR1)
                finish(2 * L, idx0, r0b, sI0, sR0)
                finish(2 * L + 1, idx1, r1b, sI1, sR1)
            if BT > 0:
                tbase = sid * ET + NB * BE
                pltpu.sync_copy(pairi_hbm.at[pl.ds(EOFF + tbase, BT)], idxt)
                pltpu.sync_copy(src_at(tbase, BT), rtb)
                pltpu.sync_copy(rtb, acc_sh.at[idxt], add=True)
            plsc.subcore_barrier()

            def write_piece(k, carry):
                r0 = (sid + 16 * k) * NP
                pltpu.sync_copy(acc_sh.at[pl.ds(r0, NP)], wbuf)
                pltpu.sync_copy(wbuf, dst_at(r0))
                return carry

            lax.fori_loop(0, npieces, write_piece, 0)

        @pl.when(cid == 0)
        def _():
            run_pass(lambda b, n: i1_hbm.at[pl.ds(b, n)],
                     lambda r: out1_hbm.at[pl.ds(r, NP)])
            run_pass(lambda b, n: ix3_hbm.at[0, pl.ds(b, n), :],
                     lambda r: out3_hbm.at[pl.ds(r, NP), pl.ds(0, F)])

        @pl.when(cid == 1)
        def _():
            run_pass(lambda b, n: ix3_hbm.at[1, pl.ds(b, n), :],
                     lambda r: out3_hbm.at[pl.ds(r, NP), pl.ds(F, F)])
            run_pass(lambda b, n: ix3_hbm.at[2, pl.ds(b, n), :],
                     lambda r: out3_hbm.at[pl.ds(r, NP), pl.ds(2 * F, F)])

    return scatter_k


def _make_node(N, F):
    Nb = 2000
    grid = N // Nb

    def body(a1_ref, a3_ref, wpp_ref, bpp_ref, weq_ref, wout_ref, bout_ref,
             p1t1_ref, p3t1_ref):
        p1n = jnp.tanh(
            jnp.dot(a1_ref[...], wpp_ref[...],
                    preferred_element_type=jnp.float32) + bpp_ref[...])
        p1t1_ref[:, 0, :] = jnp.dot(
            p1n, wout_ref[...], preferred_element_type=jnp.float32) + bout_ref[...]
        p3n = [
            jnp.dot(a3_ref[:, x * F:(x + 1) * F], weq_ref[...],
                    preferred_element_type=jnp.float32) for x in range(3)
        ]
        dot = p3n[0] * p3n[0] + p3n[1] * p3n[1] + p3n[2] * p3n[2]
        scale = jnp.dot(
            dot, wout_ref[...], preferred_element_type=jnp.float32) + bout_ref[...]
        for x in range(3):
            p3t1_ref[:, x, :] = p3n[x] * scale

    return pl.pallas_call(
        body,
        grid=(grid,),
        in_specs=[
            pl.BlockSpec((Nb, F), lambda i: (i, 0)),
            pl.BlockSpec((Nb, 3 * F), lambda i: (i, 0)),
            pl.BlockSpec((F, F), lambda i: (0, 0)),
            pl.BlockSpec((1, F), lambda i: (0, 0)),
            pl.BlockSpec((F, F), lambda i: (0, 0)),
            pl.BlockSpec((F, F), lambda i: (0, 0)),
            pl.BlockSpec((1, F), lambda i: (0, 0)),
        ],
        out_specs=[
            pl.BlockSpec((Nb, 1, F), lambda i: (i, 0, 0)),
            pl.BlockSpec((Nb, 3, F), lambda i: (i, 0, 0)),
        ],
        out_shape=[
            jax.ShapeDtypeStruct((N, 1, F), jnp.float32),
            jax.ShapeDtypeStruct((N, 3, F), jnp.float32),
        ],
    )


def kernel(p1, p3, pair_i, pair_j, basis, diff, W_pp, b_pp, W_pi, W_ii,
           W_eq_pp, W_pix, W_out, b_out):
    N, _, F = p1.shape
    E = pair_i.shape[0]
    B = basis.shape[1]

    cat = jnp.concatenate([p1.reshape(N, F), p3.reshape(N, 3 * F)], axis=1)
    W_pi_perm = W_pi.reshape(F, F, B).transpose(0, 2, 1).reshape(F, F * B)
    zeros = jnp.zeros((80, F), jnp.float32)

    s = _make_gather(N, 4 * F, 0, E)(cat, pair_i, pair_j)
    i1f, ix3 = _make_edge(E, F, B, 0)(s, basis, diff, W_pi_perm, W_ii, W_pix)
    acc1, acc3 = _make_scatter(N, F, 0, E)(i1f, ix3, pair_i, zeros)
    p1t1, p3t1 = _make_node(N, F)(
        acc1, acc3, W_pp, b_pp.reshape(1, F), W_eq_pp, W_out,
        b_out.reshape(1, F))
    return (p1t1, p3t1, i1f.reshape(E, 1, F), ix3.transpose(1, 0, 2))
